# baseline (speedup 1.0000x reference)
import functools

import jax
import jax.numpy as jnp
from jax import lax
from jax.experimental import pallas as pl
from jax.experimental.pallas import tpu as pltpu
from jax.experimental.pallas import tpu_sc as plsc

MOMENTUM = 0.99
NUM_CLASSES = 1000
B, D = 16384, 128
NC, NS = 2, 16
HALF = NUM_CLASSES // NC
HP = 512
RPT = B // NS
CH = 64
MAXCH = RPT // CH
NW = NC * NS
UROWS = HP // NS
NBUF = 3

_mesh = plsc.VectorSubcoreMesh(
    core_axis_name="c", subcore_axis_name="s", num_cores=NC, num_subcores=NS)


@functools.partial(
    pl.kernel,
    out_type=(
        jax.ShapeDtypeStruct((NW, 16), jnp.float32),
        jax.ShapeDtypeStruct((NW, 16), jnp.float32),
    ),
    mesh=_mesh,
    compiler_params=pltpu.CompilerParams(needs_layout_passes=False),
    scratch_types=dict(
        tbuf=pltpu.VMEM((RPT,), jnp.int32),
        pbuf=pltpu.VMEM((RPT,), jnp.int32),
        fidxa=pltpu.VMEM((RPT,), jnp.int32),
        sega=pltpu.VMEM((RPT,), jnp.int32),
        fidxb=pltpu.VMEM((RPT,), jnp.int32),
        segb=pltpu.VMEM((RPT,), jnp.int32),
        fidxa2=pltpu.VMEM((MAXCH, CH), jnp.int32),
        sega2=pltpu.VMEM((MAXCH, CH), jnp.int32),
        fidxb2=pltpu.VMEM((MAXCH, CH), jnp.int32),
        segb2=pltpu.VMEM((MAXCH, CH), jnp.int32),
        fb=pltpu.VMEM((NBUF, CH, D), jnp.float32),
        gb=pltpu.VMEM((NBUF, CH, D), jnp.float32),
        cbuf=pltpu.VMEM((CH, 16), jnp.float32),
        smbuf=pltpu.VMEM((UROWS, D), jnp.float32),
        cnbuf=pltpu.VMEM((UROWS, 16), jnp.float32),
        cenbuf=pltpu.VMEM((UROWS, D), jnp.float32),
        accbuf=pltpu.VMEM((16,), jnp.float32),
        obuf2=pltpu.VMEM((16,), jnp.float32),
        ssum=pltpu.VMEM_SHARED((HP, D), jnp.float32),
        scnt=pltpu.VMEM_SHARED((HP, 16), jnp.float32),
        gsems=pltpu.SemaphoreType.DMA((NBUF,)),
        hsems=pltpu.SemaphoreType.DMA((NBUF,)),
        ssems=pltpu.SemaphoreType.DMA((NBUF,)),
        csems=pltpu.SemaphoreType.DMA((NBUF,)),
    ),
)
def _center_loss_kernel(features, targets, pmarks, center, count_src,
                        zsum, zcnt, out_sq, out_np,
                        tbuf, pbuf, fidxa, sega, fidxb, segb,
                        fidxa2, sega2, fidxb2, segb2, fb, gb, cbuf,
                        smbuf, cnbuf, cenbuf, accbuf, obuf2,
                        ssum, scnt, gsems, hsems, ssems, csems):
    c = lax.axis_index("c")
    s = lax.axis_index("s")
    wid = c * NS + s
    base = s * RPT

    pltpu.sync_copy(targets.at[pl.ds(base, RPT)], tbuf)
    pltpu.sync_copy(pmarks.at[pl.ds(base, RPT)], pbuf)
    pltpu.sync_copy(count_src, cbuf)

    urows = pl.ds(s * UROWS, UROWS)
    pltpu.sync_copy(zsum.at[urows], ssum.at[urows])
    pltpu.sync_copy(zcnt.at[urows], scnt.at[urows])

    lo = c * HALF

    zi = jnp.zeros((16,), jnp.int32)
    dv = jnp.full((16,), HALF, jnp.int32)

    def prefill(k, _):
        cols = pl.ds(k * 16, 16)
        fidxa[cols] = zi
        sega[cols] = dv
        fidxb[cols] = zi
        segb[cols] = dv
        return 0

    lax.fori_loop(0, RPT // 16, prefill, 0)

    lane = lax.iota(jnp.int32, 16)

    def compact(k, carry):
        offa, offb = carry
        cols = pl.ds(k * 16, 16)
        t = tbuf[cols]
        p = pbuf[cols]
        seg = t - lo
        inhalf = (seg >= 0) & (seg < HALF)
        rowid = base + k * 16 + lane
        ma = (p == 0) & inhalf
        mb = (p != 0) & inhalf
        plsc.store_compressed(fidxa.at[pl.ds(offa, 16)], rowid, mask=ma)
        plsc.store_compressed(sega.at[pl.ds(offa, 16)], seg, mask=ma)
        plsc.store_compressed(fidxb.at[pl.ds(offb, 16)], rowid, mask=mb)
        plsc.store_compressed(segb.at[pl.ds(offb, 16)], seg, mask=mb)
        na = plsc.all_reduce_population_count(ma)[0]
        nb = plsc.all_reduce_population_count(mb)[0]
        return offa + na, offb + nb

    offa, offb = lax.fori_loop(0, RPT // 16, compact,
                               (jnp.int32(0), jnp.int32(0)))

    def repack(r, _):
        for k in range(CH // 16):
            src = pl.ds(r * CH + k * 16, 16)
            dst = pl.ds(k * 16, 16)
            fidxa2[r, dst] = fidxa[src]
            sega2[r, dst] = sega[src]
            fidxb2[r, dst] = fidxb[src]
            segb2[r, dst] = segb[src]
        return 0

    lax.fori_loop(0, MAXCH, repack, 0)

    plsc.subcore_barrier()

    ncha = (offa + (CH - 1)) // CH

    gdescs = [None] * MAXCH
    sdescs = [None] * MAXCH
    cdescs = [None] * MAXCH

    def gissue_a(j):
        gdescs[j] = pltpu.async_copy(
            features.at[fidxa2.at[j]], fb.at[j % NBUF], gsems.at[j % NBUF])

    @pl.when(0 < ncha)
    def _():
        gissue_a(0)

    for j in range(MAXCH):
        if j + 1 < MAXCH:
            @pl.when(j + 1 < ncha)
            def _(j=j):
                if j + 1 >= NBUF:
                    sdescs[j + 1 - NBUF].wait()
                    cdescs[j + 1 - NBUF].wait()
                gissue_a(j + 1)

        @pl.when(j < ncha)
        def _(j=j):
            gdescs[j].wait()
            sdescs[j] = pltpu.async_copy(
                fb.at[j % NBUF], ssum.at[sega2.at[j]], ssems.at[j % NBUF],
                add=True)
            cdescs[j] = pltpu.async_copy(
                cbuf, scnt.at[sega2.at[j]], csems.at[j % NBUF], add=True)

    for j in range(MAXCH):
        @pl.when((j < ncha) & (j >= ncha - NBUF))
        def _(j=j):
            sdescs[j].wait()
            cdescs[j].wait()

    plsc.subcore_barrier()

    pltpu.sync_copy(ssum.at[urows], smbuf)
    pltpu.sync_copy(scnt.at[urows], cnbuf)
    pltpu.sync_copy(center.at[c, pl.ds(s * UROWS, UROWS)], cenbuf)

    def update_row(r, _):
        n = cnbuf[r, pl.ds(0, 16)][0]
        has = n > 0.0
        nb = jnp.full((16,), n, jnp.float32)
        scale = (1.0 - MOMENTUM) / jnp.maximum(nb, 1.0)
        for q in range(D // 16):
            cols = pl.ds(q * 16, 16)
            sm = smbuf[r, cols]
            cen = cenbuf[r, cols]
            smbuf[r, cols] = jnp.where(has, MOMENTUM * cen + scale * sm, cen)
        return 0

    lax.fori_loop(0, UROWS, update_row, 0)
    pltpu.sync_copy(smbuf, ssum.at[urows])

    plsc.subcore_barrier()

    nchb = (offb + (CH - 1)) // CH
    accbuf[...] = jnp.zeros((16,), jnp.float32)

    fdescs = [None] * MAXCH
    hdescs = [None] * MAXCH

    def gissue_b(j):
        fdescs[j] = pltpu.async_copy(
            features.at[fidxb2.at[j]], fb.at[j % NBUF], gsems.at[j % NBUF])
        hdescs[j] = pltpu.async_copy(
            ssum.at[segb2.at[j]], gb.at[j % NBUF], hsems.at[j % NBUF])

    @pl.when(0 < nchb)
    def _():
        gissue_b(0)

    for j in range(MAXCH):
        if j + 1 < MAXCH:
            @pl.when(j + 1 < nchb)
            def _(j=j):
                gissue_b(j + 1)

        @pl.when(j < nchb)
        def _(j=j):
            fdescs[j].wait()
            hdescs[j].wait()
            fbuf = fb.at[j % NBUF]
            gbuf = gb.at[j % NBUF]

            def row_body(r, a):
                m = jnp.where(j * CH + r < offb, 1.0, 0.0)
                rs = jnp.zeros((16,), jnp.float32)
                for q in range(D // 16):
                    cols = pl.ds(q * 16, 16)
                    d = fbuf[r, cols] - gbuf[r, cols]
                    rs = rs + d * d
                return a + rs * m

            acc = lax.fori_loop(0, CH, row_body,
                                jnp.zeros((16,), jnp.float32))
            accbuf[...] = accbuf[...] + acc

    pltpu.sync_copy(accbuf, out_sq.at[wid])
    obuf2[...] = jnp.where(lane == 0, offb.astype(jnp.float32), 0.0)
    pltpu.sync_copy(obuf2, out_np.at[wid])


def kernel(features, targets, pmarks, center):
    count_src = jnp.zeros((CH, 16), jnp.float32).at[:, 0].set(1.0)
    zsum = jnp.zeros((HP, D), jnp.float32)
    zcnt = jnp.zeros((HP, 16), jnp.float32)
    center_pad = jnp.zeros((NC, HP, D), jnp.float32)
    center_pad = center_pad.at[:, :HALF].set(center.reshape(NC, HALF, D))

    out_sq, out_np = _center_loss_kernel(features, targets, pmarks,
                                         center_pad, count_src, zsum, zcnt)

    tot = jnp.sum(out_sq)
    n_p = jnp.sum(out_np)
    return tot / jnp.maximum(n_p * D, 1.0)


# speedup vs baseline: 1.8879x; 1.8879x over previous
import functools

import jax
import jax.numpy as jnp
from jax import lax
from jax.experimental import pallas as pl
from jax.experimental.pallas import tpu as pltpu
from jax.experimental.pallas import tpu_sc as plsc

MOMENTUM = 0.99
NUM_CLASSES = 1000
B, D = 16384, 128
CP = 1024
NC, NS = 2, 16
NW = NC * NS
RPW = B // NW
CH = 64
NCHS = RPW // CH
CHL = 128
NCHL = RPW // CHL
TROWS = CP // NS
NQ = D // 16
NACC = 4

_mesh = plsc.VectorSubcoreMesh(
    core_axis_name="c", subcore_axis_name="s", num_cores=NC, num_subcores=NS)


@functools.partial(
    pl.kernel,
    out_type=(
        jax.ShapeDtypeStruct((NC, CP, D), jnp.float32),
        jax.ShapeDtypeStruct((NC, CP, 16), jnp.float32),
    ),
    mesh=_mesh,
    scratch_types=dict(
        fb0=pltpu.VMEM((CH, D), jnp.float32),
        fb1=pltpu.VMEM((CH, D), jnp.float32),
        tbuf=pltpu.VMEM((RPW,), jnp.int32),
        pbuf=pltpu.VMEM((RPW,), jnp.int32),
        ibuf=pltpu.VMEM((NCHS, CH), jnp.int32),
        cbuf=pltpu.VMEM((CH, 16), jnp.float32),
        ssum=pltpu.VMEM_SHARED((CP, D), jnp.float32),
        scnt=pltpu.VMEM_SHARED((CP, 16), jnp.float32),
        sem0=pltpu.SemaphoreType.DMA,
        sem1=pltpu.SemaphoreType.DMA,
    ),
)
def _scatter_kernel(features, targets, pmarks, count_src, zsum, zcnt,
                    psum, pcnt, fb0, fb1, tbuf, pbuf, ibuf, cbuf, ssum, scnt,
                    sem0, sem1):
    c = lax.axis_index("c")
    s = lax.axis_index("s")
    wid = c * NS + s
    base = wid * RPW

    pltpu.sync_copy(targets.at[pl.ds(base, RPW)], tbuf)
    pltpu.sync_copy(pmarks.at[pl.ds(base, RPW)], pbuf)
    pltpu.sync_copy(count_src, cbuf)

    trows = pl.ds(s * TROWS, TROWS)
    pltpu.sync_copy(zsum.at[trows], ssum.at[trows])
    pltpu.sync_copy(zcnt.at[trows], scnt.at[trows])

    for k in range(RPW // 16):
        t = tbuf[pl.ds(k * 16, 16)]
        p = pbuf[pl.ds(k * 16, 16)]
        seg = jnp.where(p == 0, t, NUM_CLASSES)
        ibuf[k // (CH // 16), pl.ds((k % (CH // 16)) * 16, 16)] = seg

    plsc.subcore_barrier()

    fbs, sems, descs = (fb0, fb1), (sem0, sem1), [None, None]
    descs[0] = pltpu.async_copy(features.at[pl.ds(base, CH)], fb0, sem0)
    for j in range(NCHS):
        if j + 1 < NCHS:
            nb = (j + 1) % 2
            descs[nb] = pltpu.async_copy(
                features.at[pl.ds(base + (j + 1) * CH, CH)], fbs[nb], sems[nb])
        descs[j % 2].wait()
        pltpu.sync_copy(fbs[j % 2], ssum.at[ibuf.at[j]], add=True)
        pltpu.sync_copy(cbuf, scnt.at[ibuf.at[j]], add=True)

    plsc.subcore_barrier()

    pltpu.sync_copy(ssum.at[trows], psum.at[c, trows])
    pltpu.sync_copy(scnt.at[trows], pcnt.at[c, trows])


@functools.partial(
    pl.kernel,
    out_type=(
        jax.ShapeDtypeStruct((NW, 16), jnp.float32),
        jax.ShapeDtypeStruct((NW, 16), jnp.float32),
    ),
    mesh=_mesh,
    scratch_types=dict(
        s0buf=pltpu.VMEM((TROWS, D), jnp.float32),
        c0buf=pltpu.VMEM((TROWS, 16), jnp.float32),
        c1buf=pltpu.VMEM((TROWS, 16), jnp.float32),
        stab=pltpu.VMEM_SHARED((CP, D), jnp.float32),
        tbuf=pltpu.VMEM((NCHL, CHL), jnp.int32),
        pbuf=pltpu.VMEM((RPW,), jnp.int32),
        fb0=pltpu.VMEM((CHL, D), jnp.float32),
        fb1=pltpu.VMEM((CHL, D), jnp.float32),
        gb0=pltpu.VMEM((CHL, D), jnp.float32),
        gb1=pltpu.VMEM((CHL, D), jnp.float32),
        obuf=pltpu.VMEM((16,), jnp.float32),
        obuf2=pltpu.VMEM((16,), jnp.float32),
        fsem0=pltpu.SemaphoreType.DMA,
        fsem1=pltpu.SemaphoreType.DMA,
        gsem0=pltpu.SemaphoreType.DMA,
        gsem1=pltpu.SemaphoreType.DMA,
    ),
)
def _loss_kernel(psum, pcnt, center, features, targets, pmarks, out_sq, out_np,
                 s0buf, c0buf, c1buf, stab, tbuf, pbuf,
                 fb0, fb1, gb0, gb1, obuf, obuf2,
                 fsem0, fsem1, gsem0, gsem1):
    c = lax.axis_index("c")
    s = lax.axis_index("s")
    wid = c * NS + s
    base = wid * RPW

    trows = pl.ds(s * TROWS, TROWS)
    pltpu.sync_copy(psum.at[0, trows], s0buf)
    pltpu.sync_copy(psum.at[1, trows], gb0.at[pl.ds(0, TROWS)])
    pltpu.sync_copy(pcnt.at[0, trows], c0buf)
    pltpu.sync_copy(pcnt.at[1, trows], c1buf)
    pltpu.sync_copy(center.at[trows], gb1.at[pl.ds(0, TROWS)])

    def update_row(r, _):
        n = c0buf[r, pl.ds(0, 16)][0] + c1buf[r, pl.ds(0, 16)][0]
        has = n > 0.0
        nb = jnp.full((16,), n, jnp.float32)
        scale = (1.0 - MOMENTUM) / jnp.maximum(nb, 1.0)
        for q in range(NQ):
            cols = pl.ds(q * 16, 16)
            sm = s0buf[r, cols] + gb0[r, cols]
            cen = gb1[r, cols]
            s0buf[r, cols] = jnp.where(has, MOMENTUM * cen + scale * sm, cen)
        return 0

    lax.fori_loop(0, TROWS, update_row, 0)
    pltpu.sync_copy(s0buf, stab.at[trows])

    for j in range(NCHL):
        pltpu.sync_copy(targets.at[pl.ds(base + j * CHL, CHL)], tbuf.at[j])
    pltpu.sync_copy(pmarks.at[pl.ds(base, RPW)], pbuf)

    plsc.subcore_barrier()

    fbs, gbs = (fb0, fb1), (gb0, gb1)
    fsems, gsems = (fsem0, fsem1), (gsem0, gsem1)
    fdescs, gdescs = [None, None], [None, None]
    fdescs[0] = pltpu.async_copy(features.at[pl.ds(base, CHL)], fb0, fsem0)
    gdescs[0] = pltpu.async_copy(stab.at[tbuf.at[0]], gb0, gsem0)

    acc = jnp.zeros((16,), jnp.float32)
    npv = jnp.zeros((16,), jnp.float32)
    for j in range(NCHL):
        if j + 1 < NCHL:
            nb = (j + 1) % 2
            fdescs[nb] = pltpu.async_copy(
                features.at[pl.ds(base + (j + 1) * CHL, CHL)], fbs[nb],
                fsems[nb])
            gdescs[nb] = pltpu.async_copy(
                stab.at[tbuf.at[j + 1]], gbs[nb], gsems[nb])
        fdescs[j % 2].wait()
        gdescs[j % 2].wait()
        fbuf, gbuf = fbs[j % 2], gbs[j % 2]

        def grp_body(g, carry):
            a, nv = carry
            mv = jnp.where(pbuf[pl.ds(j * CHL + g * 16, 16)] != 0, 1.0, 0.0)
            nv = nv + mv
            for lane in range(16):
                m = mv[lane]
                for q in range(NQ):
                    cols = pl.ds(q * 16, 16)
                    d = (fbuf[g * 16 + lane, cols]
                         - gbuf[g * 16 + lane, cols]) * m
                    a = a + d * d
            return a, nv

        acc, npv = lax.fori_loop(0, CHL // 16, grp_body, (acc, npv))

    obuf[...] = acc
    pltpu.sync_copy(obuf, out_sq.at[wid])
    obuf2[...] = npv
    pltpu.sync_copy(obuf2, out_np.at[wid])


def kernel(features, targets, pmarks, center):
    count_src = jnp.zeros((CH, 16), jnp.float32).at[:, 0].set(1.0)
    zsum = jnp.zeros((CP, D), jnp.float32)
    zcnt = jnp.zeros((CP, 16), jnp.float32)
    center_pad = jnp.zeros((CP, D), jnp.float32).at[:NUM_CLASSES].set(center)

    psum, pcnt = _scatter_kernel(features, targets, pmarks, count_src,
                                 zsum, zcnt)
    out_sq, out_np = _loss_kernel(psum, pcnt, center_pad, features, targets,
                                  pmarks)

    tot = jnp.sum(out_sq)
    n_p = jnp.sum(out_np)
    return tot / jnp.maximum(n_p * D, 1.0)
